# baseline (speedup 1.0000x reference)
import functools

import jax
import jax.numpy as jnp
from jax import lax
from jax.experimental import pallas as pl
from jax.experimental.pallas import tpu as pltpu
from jax.experimental.pallas import tpu_sc as plsc

M = 1000000
D = 32
B = 16384

NC = 2
NS = 16
L = 16
NW = NC * NS

SUP = M // 4
USUP = B // 4
OCT = 976
SUP_MAX = (OCT + 1) * 8
ROWS_MAX = SUP_MAX * 4
BLK4 = 128
NBLK = (OCT * 8) // BLK4
NSEL_GROUPS = B // L

CAP = 768
UCHUNK = 64
WCAP = CAP + UCHUNK


def _body(data_hbm, idx_hbm, upd_hbm, out_hbm,
          idx_v, blist, loclist, b_w, loc_w, claim, uq, ubuf, brow,
          cbuf0, cbuf1,
          sin0, sin1, sout0, sout1, sgather):
    wid = lax.axis_index("s") * NC + lax.axis_index("c")
    lo4 = 8 * (OCT * wid + jnp.minimum(wid, 18))
    hi4 = 8 * (OCT * (wid + 1) + jnp.minimum(wid + 1, 18))
    lo = 4 * lo4
    hi = 4 * hi4

    bufs = (cbuf0, cbuf1)
    sins = (sin0, sin1)
    souts = (sout0, sout1)

    cp_in = [
        pltpu.async_copy(data_hbm.at[pl.ds(lo4, BLK4)], cbuf0, sin0),
        pltpu.async_copy(data_hbm.at[pl.ds(lo4 + BLK4, BLK4)], cbuf1, sin1),
    ]

    pltpu.sync_copy(idx_hbm, idx_v)

    lane = lax.iota(jnp.int32, L)

    def sel_body(g, off):
        idxv = idx_v[pl.ds(g * L, L)]
        m = (idxv >= lo) & (idxv < hi)
        cnt = jnp.sum(m.astype(jnp.int32))
        safe = jnp.minimum(off, CAP)
        plsc.store_compressed(blist.at[pl.ds(safe, L)], g * L + lane, mask=m)
        plsc.store_compressed(loclist.at[pl.ds(safe, L)], idxv, mask=m)
        return off + cnt

    n_sel = lax.fori_loop(0, NSEL_GROUPS, sel_body, jnp.int32(0))
    n_sel = jnp.minimum(n_sel, CAP)

    def claim_body(g, _):
        jv = g * L + lane
        valid = jv < n_sel
        locv = loclist[pl.ds(g * L, L)]
        locl = jnp.where(valid, locv - lo, 0)
        for i in range(L):
            plsc.store_scatter(claim, [locl], jv, mask=valid & (lane == i))
        return 0

    lax.fori_loop(0, (n_sel + L - 1) // L, claim_body, 0)

    def win_body(g, offw):
        jv = g * L + lane
        valid = jv < n_sel
        locv = loclist[pl.ds(g * L, L)]
        bv = blist[pl.ds(g * L, L)]
        locl = jnp.where(valid, locv - lo, 0)
        cl = plsc.load_gather(claim, [locl], mask=valid)
        win = valid & (cl == jv)
        cnt = jnp.sum(win.astype(jnp.int32))
        plsc.store_compressed(b_w.at[pl.ds(offw, L)], bv, mask=win)
        plsc.store_compressed(loc_w.at[pl.ds(offw, L)], locv, mask=win)
        return offw + cnt

    n_w = lax.fori_loop(0, (n_sel + L - 1) // L, win_body, jnp.int32(0))

    zeros = jnp.zeros((L,), jnp.int32)
    for g in range(UCHUNK // L):
        b_w[pl.ds(n_w + g * L, L)] = zeros

    nuch = (n_w + UCHUNK - 1) // UCHUNK

    def uq_body(c, _):
        base = c * UCHUNK
        for g in range(UCHUNK // L):
            brow[pl.ds(g * L, L)] = b_w[pl.ds(base + g * L, L)] >> 2
        pltpu.async_copy(upd_hbm.at[brow], ubuf, sgather).wait()
        for g in range(UCHUNK // L):
            jv = base + g * L + lane
            bv = b_w[pl.ds(base + g * L, L)]
            qcol = (bv & 3) * D
            srow = jnp.full((L,), g * L, jnp.int32) + lane
            for e in range(D):
                val = plsc.load_gather(ubuf, [srow, qcol + e])
                plsc.store_scatter(uq, [jv, jnp.full((L,), e, jnp.int32)],
                                   val)
        return 0

    lax.fori_loop(0, nuch, uq_body, 0)

    ngroups_w = (n_w + L - 1) // L

    def _patch(buf, sblk, blksz):
        def patch_body(g, _):
            jv = g * L + lane
            valid = jv < n_w
            locv = loc_w[pl.ds(g * L, L)]
            sv = locv >> 2
            inblk = valid & (sv >= sblk) & (sv < sblk + blksz)
            cnt = jnp.sum(inblk.astype(jnp.int32))

            @pl.when(cnt > 0)
            def _do():
                srow = jnp.where(inblk, sv - sblk, 0)
                qcol = (locv & 3) * D
                for e in range(D):
                    val = plsc.load_gather(uq, [jv, jnp.full((L,), e,
                                                             jnp.int32)],
                                           mask=inblk)
                    plsc.store_scatter(buf, [srow, qcol + e], val,
                                       mask=inblk)
            return 0

        lax.fori_loop(0, ngroups_w, patch_body, 0)

    def copy_pair(kk, _):
        for p in range(2):
            k = 2 * kk + p
            cp_in[p].wait()
            _patch(bufs[p], lo4 + k * BLK4, BLK4)
            cp_out = pltpu.async_copy(
                bufs[p], out_hbm.at[pl.ds(lo4 + k * BLK4, BLK4)], souts[p])
            cp_out.wait()

            @pl.when(k + 2 < NBLK)
            def _next():
                pltpu.async_copy(
                    data_hbm.at[pl.ds(lo4 + (k + 2) * BLK4, BLK4)],
                    bufs[p], sins[p])
        return 0

    lax.fori_loop(0, NBLK // 2, copy_pair, 0)

    k = NBLK - 1
    cp_in[k % 2].wait()
    _patch(bufs[k % 2], lo4 + k * BLK4, BLK4)
    pltpu.async_copy(bufs[k % 2],
                     out_hbm.at[pl.ds(lo4 + k * BLK4, BLK4)],
                     souts[k % 2]).wait()

    @pl.when(hi4 - lo4 > NBLK * BLK4)
    def _tail():
        tb = cbuf0.at[pl.ds(0, 8)]
        pltpu.sync_copy(data_hbm.at[pl.ds(lo4 + NBLK * BLK4, 8)], tb)
        _patch(cbuf0, lo4 + NBLK * BLK4, 8)
        pltpu.sync_copy(tb, out_hbm.at[pl.ds(lo4 + NBLK * BLK4, 8)])


@functools.partial(
    pl.kernel,
    out_type=jax.ShapeDtypeStruct((SUP, 4 * D), jnp.float32),
    mesh=plsc.VectorSubcoreMesh(
        core_axis_name="c", subcore_axis_name="s", num_cores=NC,
        num_subcores=NS),
    scratch_types=[
        pltpu.VMEM((B,), jnp.int32),
        pltpu.VMEM((CAP + L,), jnp.int32),
        pltpu.VMEM((CAP + L,), jnp.int32),
        pltpu.VMEM((WCAP,), jnp.int32),
        pltpu.VMEM((WCAP,), jnp.int32),
        pltpu.VMEM((ROWS_MAX,), jnp.int32),
        pltpu.VMEM((WCAP, D), jnp.float32),
        pltpu.VMEM((UCHUNK, 4 * D), jnp.float32),
        pltpu.VMEM((UCHUNK,), jnp.int32),
        pltpu.VMEM((BLK4, 4 * D), jnp.float32),
        pltpu.VMEM((BLK4, 4 * D), jnp.float32),
        pltpu.SemaphoreType.DMA,
        pltpu.SemaphoreType.DMA,
        pltpu.SemaphoreType.DMA,
        pltpu.SemaphoreType.DMA,
        pltpu.SemaphoreType.DMA,
    ],
    compiler_params=pltpu.CompilerParams(
        needs_layout_passes=False, use_tc_tiling_on_sc=False),
)
def _scatter_nd_sc(data_hbm, idx_hbm, upd_hbm, out_hbm, *scratch):
    _body(data_hbm, idx_hbm, upd_hbm, out_hbm, *scratch)


def kernel(data, indices, updates):
    data4 = data.reshape(SUP, 4 * D)
    upd4 = updates.reshape(USUP, 4 * D)
    out4 = _scatter_nd_sc(data4, indices.reshape(B), upd4)
    return out4.reshape(M, D)
